# Initial kernel scaffold; baseline (speedup 1.0000x reference)
import jax
import jax.numpy as jnp
from jax.experimental import pallas as pl


def kernel(x, edge_index, edge_weight, W):
    raise NotImplementedError("write your pallas kernel here")


# speedup vs baseline: 6.5729x; 6.5729x over previous
import functools

import jax
import jax.numpy as jnp
from jax import lax
from jax.experimental import pallas as pl
from jax.experimental.pallas import tpu as pltpu
from jax.experimental.pallas import tpu_sc as plsc

_NC = 2
_NS = 16
_NW = _NC * _NS
_L = 16


def _matmul_body(x_ref, w_ref, h_ref):
    h_ref[...] = jnp.dot(x_ref[...], w_ref[...],
                         preferred_element_type=jnp.float32)


def _combine_body(p_ref, h_ref, o_ref):
    o_ref[...] = (p_ref[0] + p_ref[1] + h_ref[...]) * 0.5


def kernel(x, edge_index, edge_weight, W):
    N, D = x.shape
    U = W.shape[1]
    E = edge_index.shape[1]

    MB = 1000

    h = pl.pallas_call(
        _matmul_body,
        grid=(N // MB,),
        in_specs=[pl.BlockSpec((MB, D), lambda i: (i, 0)),
                  pl.BlockSpec((D, U), lambda i: (0, 0))],
        out_specs=pl.BlockSpec((MB, U), lambda i: (i, 0)),
        out_shape=jax.ShapeDtypeStruct((N, U), jnp.float32),
    )(x, W)

    EPW = E // _NW
    B = 80
    NBLK = EPW // B
    RSTRIDE = 624
    RPT = 640
    ZR = 80
    ZCH = RPT // ZR

    mesh = plsc.VectorSubcoreMesh(core_axis_name="c", subcore_axis_name="s")

    @functools.partial(
        pl.kernel,
        out_type=jax.ShapeDtypeStruct((_NC, N, U), jnp.float32),
        mesh=mesh,
        scratch_types=[
            pltpu.VMEM((EPW,), jnp.int32),
            pltpu.VMEM((EPW,), jnp.int32),
            pltpu.VMEM((EPW,), jnp.float32),
            pltpu.VMEM((B,), jnp.int32),
            pltpu.VMEM((B,), jnp.int32),
            pltpu.VMEM((B, U), jnp.float32),
            pltpu.VMEM((ZR, U), jnp.float32),
            pltpu.VMEM_SHARED((N, U), jnp.float32),
            pltpu.SemaphoreType.DMA,
        ],
    )
    def _edge_kernel(h_hbm, src_hbm, dst_hbm, ew_hbm, part_hbm,
                     src_all, dst_all, w_all, src_v, dst_v, rows_v,
                     zeros_v, acc_sh, sem):
        c = lax.axis_index("c")
        s = lax.axis_index("s")
        wid = s * _NC + c

        def _zrow(i, carry):
            for j in range(U // _L):
                zeros_v[i, pl.ds(j * _L, _L)] = jnp.zeros((_L,), jnp.float32)
            return carry
        lax.fori_loop(0, ZR, _zrow, 0)
        r0 = s * RSTRIDE
        for k in range(ZCH):
            pltpu.sync_copy(zeros_v, acc_sh.at[pl.ds(r0 + k * ZR, ZR)])
        plsc.subcore_barrier()

        base = wid * EPW
        pltpu.sync_copy(src_hbm.at[pl.ds(base, EPW)], src_all)
        pltpu.sync_copy(dst_hbm.at[pl.ds(base, EPW)], dst_all)
        pltpu.sync_copy(ew_hbm.at[pl.ds(base, EPW)], w_all)

        def _blk(g, carry):
            off = g * B
            for i in range(B // _L):
                sl = pl.ds(i * _L, _L)
                src_v[sl] = src_all[pl.ds(off + i * _L, _L)]
                dst_v[sl] = dst_all[pl.ds(off + i * _L, _L)]
            pltpu.async_copy(h_hbm.at[src_v], rows_v, sem).wait()

            def _e16(t, ecarry):
                wv = w_all[pl.ds(off + t * _L, _L)]
                for i in range(_L):
                    w = wv[i]
                    b = t * _L + i
                    for j in range(U // _L):
                        sl = pl.ds(j * _L, _L)
                        rows_v[b, sl] = rows_v[b, sl] * w
                return ecarry
            lax.fori_loop(0, B // _L, _e16, 0)

            pltpu.sync_copy(rows_v, acc_sh.at[dst_v], add=True)
            return carry
        lax.fori_loop(0, NBLK, _blk, 0)

        plsc.subcore_barrier()
        pltpu.sync_copy(acc_sh.at[pl.ds(r0, RPT)],
                        part_hbm.at[c, pl.ds(r0, RPT)])

    parts = _edge_kernel(h, edge_index[1], edge_index[0], edge_weight)

    out = pl.pallas_call(
        _combine_body,
        grid=(N // MB,),
        in_specs=[pl.BlockSpec((_NC, MB, U), lambda i: (0, i, 0)),
                  pl.BlockSpec((MB, U), lambda i: (i, 0))],
        out_specs=pl.BlockSpec((MB, U), lambda i: (i, 0)),
        out_shape=jax.ShapeDtypeStruct((N, U), jnp.float32),
    )(parts, h)
    return out
